# baseline (speedup 1.0000x reference)
import functools

import jax
import jax.numpy as jnp
from jax import lax
from jax.experimental import pallas as pl
from jax.experimental.pallas import tpu as pltpu
from jax.experimental.pallas import tpu_sc as plsc

_B, _S, _C, _T = 16, 4, 2, 64000
_NROWS = _B * _S
_NC, _NS = 2, 16
_NW = _NC * _NS
_RPW = _NROWS // _NW
_CW = 32000
_CPR = _T // _CW
_CPW = _RPW * _CPR
_NBUF = 2
_LOOKAHEAD = 1

_SRC_ROWS = [
    4, 9, 22, 63, 12, 61, 30, 19, 36, 41, 42, 23, 44, 1, 2, 15,
    20, 17, 6, 11, 60, 45, 18, 43, 0, 49, 10, 47, 56, 21, 54, 51,
    8, 29, 50, 31, 48, 37, 26, 27, 24, 53, 14, 3, 28, 25, 34, 59,
    52, 13, 46, 55, 40, 57, 58, 7, 16, 5, 62, 39, 32, 33, 38, 35,
]


def _make_shuffle():
    mesh = plsc.VectorSubcoreMesh(
        core_axis_name="c", subcore_axis_name="s",
        num_cores=_NC, num_subcores=_NS,
    )

    @functools.partial(
        pl.kernel,
        out_type=jax.ShapeDtypeStruct((_B, _S, _C, _T), jnp.float32),
        mesh=mesh,
        scratch_types=(
            [pltpu.VMEM((_NBUF, _C, _CW), jnp.float32)]
            + [pltpu.SemaphoreType.DMA] * (2 * _NBUF)
        ),
        compiler_params=pltpu.CompilerParams(
            use_tc_tiling_on_sc=True, skip_device_barrier=True),
    )
    def shuffle(x_hbm, out_hbm, buf, *sems):
        in_sems = sems[:_NBUF]
        out_sems = sems[_NBUF:]
        wid = lax.axis_index("s") * _NC + lax.axis_index("c")

        packed = [_SRC_ROWS[w * _RPW] + 64 * _SRC_ROWS[w * _RPW + 1]
                  for w in range(_NW)]
        acc = jnp.int32(packed[0])
        for w in range(1, _NW):
            acc = jnp.where(wid == w, jnp.int32(packed[w]), acc)
        src = [acc % 64, acc // 64]
        dst0 = wid * _RPW

        def src_slice(k):
            r_local, c = divmod(k, _CPR)
            return x_hbm.at[src[r_local] // _S, src[r_local] % _S,
                            :, pl.ds(c * _CW, _CW)]

        def dst_slice(k):
            r_local, c = divmod(k, _CPR)
            r = dst0 + r_local
            return out_hbm.at[r // _S, r % _S, :, pl.ds(c * _CW, _CW)]

        in_cps = [None] * _CPW
        out_cps = [None] * _CPW
        out_waited = [False] * _CPW

        def start_in(j):
            cp = pltpu.make_async_copy(
                src_slice(j), buf.at[j % _NBUF], in_sems[j % _NBUF])
            cp.start()
            in_cps[j] = cp

        for j in range(min(_LOOKAHEAD, _CPW)):
            start_in(j)
        for k in range(_CPW):
            in_cps[k].wait()
            cp = pltpu.make_async_copy(
                buf.at[k % _NBUF], dst_slice(k), out_sems[k % _NBUF])
            cp.start()
            out_cps[k] = cp
            j = k + _LOOKAHEAD
            if j < _CPW:
                if j - _NBUF >= 0:
                    out_cps[j - _NBUF].wait()
                    out_waited[j - _NBUF] = True
                start_in(j)
        for k in range(_CPW):
            if not out_waited[k]:
                out_cps[k].wait()

    return shuffle


_shuffle = _make_shuffle()


def kernel(signals):
    return _shuffle(signals)


# speedup vs baseline: 1.0307x; 1.0307x over previous
import functools

import jax
import jax.numpy as jnp
from jax import lax
from jax.experimental import pallas as pl
from jax.experimental.pallas import tpu as pltpu
from jax.experimental.pallas import tpu_sc as plsc

_B, _S, _C, _T = 16, 4, 2, 64000
_NROWS = _B * _S
_NC, _NS = 2, 16
_NW = _NC * _NS
_RPW = _NROWS // _NW
_CW = 16000
_CPR = _T // _CW
_CPW = _RPW * _CPR
_NBUF = 4
_LOOKAHEAD = 2

_SRC_ROWS = [
    4, 9, 22, 63, 12, 61, 30, 19, 36, 41, 42, 23, 44, 1, 2, 15,
    20, 17, 6, 11, 60, 45, 18, 43, 0, 49, 10, 47, 56, 21, 54, 51,
    8, 29, 50, 31, 48, 37, 26, 27, 24, 53, 14, 3, 28, 25, 34, 59,
    52, 13, 46, 55, 40, 57, 58, 7, 16, 5, 62, 39, 32, 33, 38, 35,
]


def _make_shuffle():
    mesh = plsc.VectorSubcoreMesh(
        core_axis_name="c", subcore_axis_name="s",
        num_cores=_NC, num_subcores=_NS,
    )

    @functools.partial(
        pl.kernel,
        out_type=jax.ShapeDtypeStruct((_B, _S, _C, _T), jnp.float32),
        mesh=mesh,
        scratch_types=(
            [pltpu.VMEM((_NBUF, _C, _CW), jnp.float32)]
            + [pltpu.SemaphoreType.DMA] * (2 * _NBUF)
        ),
        compiler_params=pltpu.CompilerParams(
            use_tc_tiling_on_sc=True, skip_device_barrier=True),
    )
    def shuffle(x_hbm, out_hbm, buf, *sems):
        in_sems = sems[:_NBUF]
        out_sems = sems[_NBUF:]
        wid = lax.axis_index("s") * _NC + lax.axis_index("c")

        packed = [_SRC_ROWS[w * _RPW] + 64 * _SRC_ROWS[w * _RPW + 1]
                  for w in range(_NW)]
        acc = jnp.int32(packed[0])
        for w in range(1, _NW):
            acc = jnp.where(wid == w, jnp.int32(packed[w]), acc)
        src = [acc % 64, acc // 64]
        dst0 = wid * _RPW

        def src_slice(k):
            r_local, c = divmod(k, _CPR)
            return x_hbm.at[src[r_local] // _S, src[r_local] % _S,
                            :, pl.ds(c * _CW, _CW)]

        def dst_slice(k):
            r_local, c = divmod(k, _CPR)
            r = dst0 + r_local
            return out_hbm.at[r // _S, r % _S, :, pl.ds(c * _CW, _CW)]

        in_cps = [None] * _CPW
        out_cps = [None] * _CPW
        out_waited = [False] * _CPW

        def start_in(j):
            cp = pltpu.make_async_copy(
                src_slice(j), buf.at[j % _NBUF], in_sems[j % _NBUF])
            cp.start()
            in_cps[j] = cp

        for j in range(min(_LOOKAHEAD, _CPW)):
            start_in(j)
        for k in range(_CPW):
            in_cps[k].wait()
            cp = pltpu.make_async_copy(
                buf.at[k % _NBUF], dst_slice(k), out_sems[k % _NBUF])
            cp.start()
            out_cps[k] = cp
            j = k + _LOOKAHEAD
            if j < _CPW:
                if j - _NBUF >= 0:
                    out_cps[j - _NBUF].wait()
                    out_waited[j - _NBUF] = True
                start_in(j)
        for k in range(_CPW):
            if not out_waited[k]:
                out_cps[k].wait()

    return shuffle


_shuffle = _make_shuffle()


def kernel(signals):
    return _shuffle(signals)
